# Initial kernel scaffold; baseline (speedup 1.0000x reference)
import jax
import jax.numpy as jnp
from jax.experimental import pallas as pl


def kernel(x):
    raise NotImplementedError("write your pallas kernel here")


# speedup vs baseline: 25.2382x; 25.2382x over previous
import functools

import jax
import jax.numpy as jnp
from jax import lax
from jax.experimental import pallas as pl
from jax.experimental.pallas import tpu as pltpu
from jax.experimental.pallas import tpu_sc as plsc

_N = 2048
_ROWS = 8
_BETA = 1.0
_KCHUNK = 256


def _rank_body(x_ref, xc_ref, out_ref):
    a_row = jnp.abs(x_ref[0])
    a_col = jnp.abs(xc_ref[0])
    for c in range(_N // _KCHUNK):
        keys = a_col[c * _KCHUNK:(c + 1) * _KCHUNK, :]
        gt = a_row > keys
        eq = a_row == keys
        ii = lax.broadcasted_iota(jnp.int32, (_KCHUNK, _N), 0) + c * _KCHUNK
        jj = lax.broadcasted_iota(jnp.int32, (_KCHUNK, _N), 1)
        before = jnp.logical_or(gt, jnp.logical_and(eq, jj < ii))
        cnt = jnp.sum(before.astype(jnp.int32), axis=1, keepdims=True)
        out_ref[0, c * _KCHUNK:(c + 1) * _KCHUNK, :] = cnt


def _ranks_tc(x):
    x3 = x.reshape(_ROWS, 1, _N)
    xc = x.reshape(_ROWS, _N, 1)
    return pl.pallas_call(
        _rank_body,
        grid=(_ROWS,),
        in_specs=[
            pl.BlockSpec((1, 1, _N), lambda r: (r, 0, 0)),
            pl.BlockSpec((1, _N, 1), lambda r: (r, 0, 0)),
        ],
        out_specs=pl.BlockSpec((1, _N, 1), lambda r: (r, 0, 0)),
        out_shape=jax.ShapeDtypeStruct((_ROWS, _N, 1), jnp.int32),
    )(x3, xc)


def _sc_prox(x, rank, recip):
    mesh = plsc.VectorSubcoreMesh(core_axis_name="c", subcore_axis_name="s")

    @functools.partial(
        pl.kernel,
        mesh=mesh,
        out_type=jax.ShapeDtypeStruct((_ROWS, _N), jnp.float32),
        compiler_params=pltpu.CompilerParams(
            use_tc_tiling_on_sc=False, needs_layout_passes=False),
        scratch_types=[
            pltpu.VMEM((_N,), jnp.float32),
            pltpu.VMEM((_N,), jnp.int32),
            pltpu.VMEM((_N,), jnp.float32),
            pltpu.VMEM((_N + 16,), jnp.float32),
            pltpu.VMEM((_N + 16,), jnp.float32),
            pltpu.VMEM((_N,), jnp.float32),
            pltpu.VMEM((_N,), jnp.float32),
            pltpu.VMEM((_N + 16,), jnp.float32),
        ],
    )
    def k(x_hbm, rank_hbm, recip_hbm, z_hbm, v_ref, r_ref, s_ref, pm_ref,
          pc_ref, y_ref, z_ref, recip_ref):
        wid = lax.axis_index("s") * 2 + lax.axis_index("c")

        def sload(ref, idx):
            return plsc.load_gather(ref, [jnp.full((16,), idx, jnp.int32)])[0]

        def sstore(ref, idx, val):
            plsc.store_scatter(ref, [jnp.full((16,), idx, jnp.int32)],
                               jnp.full((16,), val, ref.dtype))

        @pl.when(wid < _ROWS)
        def _():
            row = wid
            pltpu.sync_copy(x_hbm.at[row], v_ref)
            pltpu.sync_copy(rank_hbm.at[row], r_ref)
            pltpu.sync_copy(recip_hbm, recip_ref)

            def scat(p, _):
                r = r_ref[pl.ds(p * 16, 16)]
                a = jnp.abs(v_ref[pl.ds(p * 16, 16)])
                s = a - _BETA * (float(_N - 1) - r.astype(jnp.float32))
                plsc.store_scatter(s_ref, [r], s)
                return 0

            lax.fori_loop(0, _N // 16, scat, 0)

            big = jnp.float32(jnp.inf)

            def pav_vreg(p, carry):
                sv = s_ref[pl.ds(p * 16, 16)]
                for lane in range(16):
                    d, tm, tc = carry
                    cm = sv[lane]
                    cc = jnp.float32(1.0)

                    def cond(st):
                        _d, ttm, _tc, m, _c = st
                        return ttm <= m

                    def merge(st):
                        dd, ttm, ttc, m, c = st
                        c2 = ttc + c
                        m2 = (ttm * ttc + m * c) * sload(
                            recip_ref, c2.astype(jnp.int32))
                        dd = dd - 1
                        ntm = sload(pm_ref, dd - 1)
                        ntc = sload(pc_ref, dd - 1)
                        return (dd, ntm, ntc, m2, c2)

                    d, tm, tc, cm, cc = lax.while_loop(
                        cond, merge, (d, tm, tc, cm, cc))
                    sstore(pm_ref, d - 1, tm)
                    sstore(pc_ref, d - 1, tc)
                    carry = (d + 1, cm, cc)
                return carry

            d, tm, tc = lax.fori_loop(
                0, _N // 16, pav_vreg, (jnp.int32(1), big, jnp.float32(1.0)))
            sstore(pm_ref, d - 1, tm)
            sstore(pc_ref, d - 1, tc)

            def init_y(p, _):
                y_ref[pl.ds(p * 16, 16)] = jnp.full((16,), big, jnp.float32)
                return 0

            lax.fori_loop(0, _N // 16, init_y, 0)

            lane_iota = lax.broadcasted_iota(jnp.int32, (16,), 0)

            def scatter_pools(p, start_carry):
                slot = lane_iota + p * 16
                valid = jnp.logical_and(slot >= 1, slot < d)
                pcv = jnp.where(valid, pc_ref[pl.ds(p * 16, 16)], 0.0)
                pmv = pm_ref[pl.ds(p * 16, 16)]
                incl = plsc.cumsum(pcv)
                starts = (start_carry + incl - pcv).astype(jnp.int32)
                plsc.store_scatter(y_ref, [starts], pmv, mask=valid)
                return start_carry + incl[15]

            lax.fori_loop(0, (_N + 16) // 16, scatter_pools, jnp.float32(0.0))

            def fill(p, neg_carry):
                yv = y_ref[pl.ds(p * 16, 16)]
                m = plsc.cummax(-yv)
                m = jnp.maximum(m, neg_carry)
                y_ref[pl.ds(p * 16, 16)] = -m
                return m[15]

            lax.fori_loop(0, _N // 16, fill, -big)

            def gath(p, _):
                r = r_ref[pl.ds(p * 16, 16)]
                yv = jnp.maximum(plsc.load_gather(y_ref, [r]), 0.0)
                sg = jnp.sign(v_ref[pl.ds(p * 16, 16)])
                z_ref[pl.ds(p * 16, 16)] = sg * yv
                return 0

            lax.fori_loop(0, _N // 16, gath, 0)
            pltpu.sync_copy(z_ref, z_hbm.at[row])

    return k(x, rank, recip)


def _sparsemax_body(z_ref, o_ref):
    z = z_ref[...]
    zmax = jnp.max(z, axis=1, keepdims=True)
    lo = zmax - 1.0
    hi = zmax

    def it(_, lohi):
        lo, hi = lohi
        mid = 0.5 * (lo + hi)
        f = jnp.sum(jnp.maximum(z - mid, 0.0), axis=1, keepdims=True)
        pred = f >= 1.0
        return (jnp.where(pred, mid, lo), jnp.where(pred, hi, mid))

    lo, hi = lax.fori_loop(0, 48, it, (lo, hi))
    tau = 0.5 * (lo + hi)
    o_ref[...] = jnp.maximum(z - tau, 0.0)


def _sparsemax_tc(z):
    return pl.pallas_call(
        _sparsemax_body,
        out_shape=jax.ShapeDtypeStruct((_ROWS, _N), jnp.float32),
    )(z)


def kernel(x):
    rank = _ranks_tc(x).reshape(_ROWS, _N)
    recip = 1.0 / jnp.maximum(
        jnp.arange(_N + 16, dtype=jnp.float32), 1.0)
    z = _sc_prox(x, rank, recip)
    return _sparsemax_tc(z)


# speedup vs baseline: 50.6837x; 2.0082x over previous
import functools

import jax
import jax.numpy as jnp
from jax import lax
from jax.experimental import pallas as pl
from jax.experimental.pallas import tpu as pltpu
from jax.experimental.pallas import tpu_sc as plsc

_N = 2048
_ROWS = 8
_BETA = 1.0
_KCHUNK = 256


def _rank_body(x_ref, xc_ref, out_ref):
    a_row = jnp.abs(x_ref[0])
    a_col = jnp.abs(xc_ref[0])
    for c in range(_N // _KCHUNK):
        keys = a_col[c * _KCHUNK:(c + 1) * _KCHUNK, :]
        gt = a_row > keys
        eq = a_row == keys
        ii = lax.broadcasted_iota(jnp.int32, (_KCHUNK, _N), 0) + c * _KCHUNK
        jj = lax.broadcasted_iota(jnp.int32, (_KCHUNK, _N), 1)
        before = jnp.logical_or(gt, jnp.logical_and(eq, jj < ii))
        cnt = jnp.sum(before.astype(jnp.int32), axis=1, keepdims=True)
        out_ref[0, c * _KCHUNK:(c + 1) * _KCHUNK, :] = cnt


def _ranks_tc(x):
    x3 = x.reshape(_ROWS, 1, _N)
    xc = x.reshape(_ROWS, _N, 1)
    return pl.pallas_call(
        _rank_body,
        grid=(_ROWS,),
        in_specs=[
            pl.BlockSpec((1, 1, _N), lambda r: (r, 0, 0)),
            pl.BlockSpec((1, _N, 1), lambda r: (r, 0, 0)),
        ],
        out_specs=pl.BlockSpec((1, _N, 1), lambda r: (r, 0, 0)),
        out_shape=jax.ShapeDtypeStruct((_ROWS, _N, 1), jnp.int32),
    )(x3, xc)


def _sc_oscarmax(x, rank, recip):
    mesh = plsc.VectorSubcoreMesh(core_axis_name="c", subcore_axis_name="s")
    nv = _N // 16

    @functools.partial(
        pl.kernel,
        mesh=mesh,
        out_type=jax.ShapeDtypeStruct((_ROWS, _N), jnp.float32),
        compiler_params=pltpu.CompilerParams(
            use_tc_tiling_on_sc=False, needs_layout_passes=False),
        scratch_types=[
            pltpu.VMEM((_N,), jnp.float32),
            pltpu.VMEM((_N,), jnp.int32),
            pltpu.VMEM((_N,), jnp.float32),
            pltpu.VMEM((_N,), jnp.float32),
            pltpu.VMEM((_N,), jnp.float32),
            pltpu.VMEM((_N + 32,), jnp.int32),
            pltpu.VMEM((_N + 16,), jnp.float32),
            pltpu.VMEM((_N + 16,), jnp.float32),
            pltpu.VMEM((_N + 16,), jnp.float32),
            pltpu.VMEM((_N + 16,), jnp.float32),
            pltpu.VMEM((_N,), jnp.float32),
            pltpu.VMEM((_N,), jnp.float32),
            pltpu.VMEM((_N,), jnp.float32),
            pltpu.VMEM((_N + 16,), jnp.float32),
        ],
    )
    def k(x_hbm, rank_hbm, recip_hbm, out_hbm, v_ref, r_ref, a_ref, sg_ref,
          ca_ref, st_ref, rm_ref, rc_ref, pm_ref, pc_ref, y_ref, zs_ref,
          o_ref, recip_ref):
        wid = lax.axis_index("s") * 2 + lax.axis_index("c")
        lane = lax.broadcasted_iota(jnp.int32, (16,), 0)
        inf = jnp.float32(jnp.inf)

        def sload(ref, idx):
            return plsc.load_gather(ref, [jnp.full((16,), idx, jnp.int32)])[0]

        def sstore(ref, idx, val):
            plsc.store_scatter(ref, [jnp.full((16,), idx, jnp.int32)],
                               jnp.full((16,), val, ref.dtype))

        @pl.when(wid < _ROWS)
        def _():
            row = wid
            pltpu.sync_copy(x_hbm.at[row], v_ref)
            pltpu.sync_copy(rank_hbm.at[row], r_ref)
            pltpu.sync_copy(recip_hbm, recip_ref)

            def scat(p, npos):
                r = r_ref[pl.ds(p * 16, 16)]
                xv = v_ref[pl.ds(p * 16, 16)]
                plsc.store_scatter(a_ref, [r], jnp.abs(xv))
                plsc.store_scatter(sg_ref, [r], jnp.sign(xv))
                return npos + jnp.sum((xv > 0).astype(jnp.float32))

            npos = lax.fori_loop(0, nv, scat, jnp.float32(0.0))

            def runscan(p, carry):
                cA, nbrk = carry
                av = a_ref[pl.ds(p * 16, 16)]
                incl = plsc.cumsum(av) + cA
                ca_ref[pl.ds(p * 16, 16)] = incl
                gpos = lane + p * 16
                prev = plsc.load_gather(a_ref, [jnp.maximum(gpos - 1, 0)])
                prev = jnp.where(gpos == 0, inf, prev)
                brk = (prev - av) > jnp.float32(_BETA)
                bf = brk.astype(jnp.float32)
                binc = plsc.cumsum(bf)
                tgt = nbrk + (binc - bf).astype(jnp.int32)
                plsc.store_scatter(st_ref, [tgt], gpos, mask=brk)
                return (incl[15], nbrk + binc[15].astype(jnp.int32))

            _, nrun = lax.fori_loop(0, nv, runscan, (jnp.float32(0.0),
                                                     jnp.int32(0)))
            sstore(st_ref, nrun, jnp.int32(_N))

            def runstat(q, _):
                base = q * 16
                idx = lane + base
                valid = idx < nrun
                cidx = jnp.where(valid, idx, 0)
                b = plsc.load_gather(st_ref, [cidx])
                e = plsc.load_gather(st_ref, [cidx + 1])
                bf = b.astype(jnp.float32)
                ef = e.astype(jnp.float32)
                cb = jnp.where(b == 0, 0.0,
                               plsc.load_gather(ca_ref,
                                                [jnp.maximum(b - 1, 0)]))
                ce = plsc.load_gather(ca_ref, [jnp.maximum(e - 1, 0)])
                cnt = ef - bf
                sum_a = ce - cb
                sum_w = _BETA * (cnt * float(_N - 1)
                                 - (bf + ef - 1.0) * cnt * 0.5)
                rcp = plsc.load_gather(
                    recip_ref, [jnp.where(valid, e - b, 1)])
                rm_ref[pl.ds(base, 16)] = (sum_a - sum_w) * rcp
                rc_ref[pl.ds(base, 16)] = jnp.where(valid, cnt, 0.0)
                return 0

            nvq = (nrun + 15) >> 4
            lax.fori_loop(0, nvq, runstat, 0)

            def pav(q, carry):
                d, tm, tc = carry
                cm = sload(rm_ref, q)
                cc = sload(rc_ref, q)

                def cond(st):
                    _d, ttm, _tc, m, _c = st
                    return ttm <= m

                def merge(st):
                    dd, ttm, ttc, m, c = st
                    c2 = ttc + c
                    m2 = (ttm * ttc + m * c) * sload(
                        recip_ref, c2.astype(jnp.int32))
                    dd = dd - 1
                    return (dd, sload(pm_ref, dd - 1), sload(pc_ref, dd - 1),
                            m2, c2)

                d, tm, tc, cm, cc = lax.while_loop(
                    cond, merge, (d, tm, tc, cm, cc))
                sstore(pm_ref, d - 1, tm)
                sstore(pc_ref, d - 1, tc)
                return (d + 1, cm, cc)

            d, tm, tc = lax.fori_loop(
                0, nrun, pav, (jnp.int32(1), inf, jnp.float32(1.0)))
            sstore(pm_ref, d - 1, tm)
            sstore(pc_ref, d - 1, tc)

            def init_y(p, _):
                y_ref[pl.ds(p * 16, 16)] = jnp.full((16,), inf, jnp.float32)
                return 0

            lax.fori_loop(0, nv, init_y, 0)

            def scatter_pools(p, start_carry):
                slot = lane + p * 16
                valid = jnp.logical_and(slot >= 1, slot < d)
                pcv = jnp.where(valid, pc_ref[pl.ds(p * 16, 16)], 0.0)
                pmv = pm_ref[pl.ds(p * 16, 16)]
                incl = plsc.cumsum(pcv)
                starts = (start_carry + incl - pcv).astype(jnp.int32)
                plsc.store_scatter(y_ref, [starts], pmv, mask=valid)
                return start_carry + incl[15]

            lax.fori_loop(0, (d + 15) >> 4, scatter_pools, jnp.float32(0.0))

            def fill(p, neg_carry):
                yv = y_ref[pl.ds(p * 16, 16)]
                m = jnp.maximum(plsc.cummax(-yv), neg_carry)
                y_ref[pl.ds(p * 16, 16)] = -m
                return m[15]

            lax.fori_loop(0, nv, fill, -inf)

            def build_zs(p, carry):
                cpos, czer, cneg = carry
                yc = jnp.maximum(y_ref[pl.ds(p * 16, 16)], 0.0)
                sgv = sg_ref[pl.ds(p * 16, 16)]
                fp = (sgv > 0).astype(jnp.float32)
                fz = (sgv == 0).astype(jnp.float32)
                fn = (sgv < 0).astype(jnp.float32)
                ip_ = plsc.cumsum(fp)
                iz = plsc.cumsum(fz)
                in_ = plsc.cumsum(fn)
                tp = cpos + (ip_ - fp)
                tz = npos + czer + (iz - fz)
                tn = float(_N - 1) - (cneg + (in_ - fn))
                tgt = (fp * tp + fz * tz + fn * tn).astype(jnp.int32)
                val = (fp - fn) * yc
                plsc.store_scatter(zs_ref, [tgt], val)
                return (cpos + ip_[15], czer + iz[15], cneg + in_[15])

            lax.fori_loop(0, nv, build_zs,
                          (jnp.float32(0.0), jnp.float32(0.0),
                           jnp.float32(0.0)))

            def smax(p, carry):
                cs, ssum, scnt = carry
                zv = zs_ref[pl.ds(p * 16, 16)]
                ics = plsc.cumsum(zv) + cs
                kk = (lane + p * 16 + 1).astype(jnp.float32)
                sup = (1.0 + kk * zv) > ics
                sf = sup.astype(jnp.float32)
                ssum = ssum + jnp.sum(jnp.where(sup, zv, 0.0))
                scnt = scnt + jnp.sum(sf)
                return (ics[15], ssum, scnt)

            _, ssum, scnt = lax.fori_loop(
                0, nv, smax, (jnp.float32(0.0), jnp.float32(0.0),
                              jnp.float32(0.0)))
            k_z = jnp.maximum(scnt, 1.0)
            tau = (ssum - 1.0) * sload(recip_ref, k_z.astype(jnp.int32))

            def outp(p, _):
                r = r_ref[pl.ds(p * 16, 16)]
                yv = jnp.maximum(plsc.load_gather(y_ref, [r]), 0.0)
                sg = jnp.sign(v_ref[pl.ds(p * 16, 16)])
                o_ref[pl.ds(p * 16, 16)] = jnp.maximum(sg * yv - tau, 0.0)
                return 0

            lax.fori_loop(0, nv, outp, 0)
            pltpu.sync_copy(o_ref, out_hbm.at[row])

    return k(x, rank, recip)


def kernel(x):
    rank = _ranks_tc(x).reshape(_ROWS, _N)
    recip = 1.0 / jnp.maximum(
        jnp.arange(_N + 16, dtype=jnp.float32), 1.0)
    return _sc_oscarmax(x, rank, recip)


# speedup vs baseline: 68.3913x; 1.3494x over previous
import functools

import numpy as np

import jax
import jax.numpy as jnp
from jax import lax
from jax.experimental import pallas as pl
from jax.experimental.pallas import tpu as pltpu
from jax.experimental.pallas import tpu_sc as plsc

_N = 2048
_ROWS = 8
_BETA = 1.0
_KCHUNK = 256


def _rank_body(x_ref, out_ref):
    k_all = lax.bitcast_convert_type(jnp.abs(x_ref[...]), jnp.int32)
    kT = jnp.transpose(k_all)
    nchunk = _N // _KCHUNK
    jlt_diag = (lax.broadcasted_iota(jnp.int32, (_KCHUNK, _KCHUNK), 1)
                < lax.broadcasted_iota(jnp.int32, (_KCHUNK, _KCHUNK), 0)
                ).astype(jnp.int32)
    for r in range(_ROWS):
        kr = k_all[r:r + 1, :]
        kc = kT[:, r:r + 1]
        col_parts = []
        colacc = [None] * nchunk
        for i_ in range(nchunk):
            kcol = kc[i_ * _KCHUNK:(i_ + 1) * _KCHUNK, :]
            acc2d = None
            for j_ in range(i_, nchunk):
                krow = kr[:, j_ * _KCHUNK:(j_ + 1) * _KCHUNK]
                if j_ == i_:
                    t = ((krow + jlt_diag) > kcol).astype(jnp.int32)
                else:
                    t = (krow > kcol).astype(jnp.int32)
                acc2d = t if acc2d is None else acc2d + t
                if j_ > i_:
                    colacc[j_] = t if colacc[j_] is None else colacc[j_] + t
            col_parts.append(jnp.sum(acc2d, axis=1, keepdims=True))
        row_parts = [
            jnp.zeros((1, _KCHUNK), jnp.int32) if colacc[j_] is None
            else (j_ * _KCHUNK
                  - jnp.sum(colacc[j_], axis=0, keepdims=True))
            for j_ in range(nchunk)
        ]
        col_full = jnp.concatenate(col_parts, axis=0)
        out_ref[r:r + 1, :] = (jnp.transpose(col_full)
                               + jnp.concatenate(row_parts, axis=1))


def _ranks_tc(x):
    return pl.pallas_call(
        _rank_body,
        out_shape=jax.ShapeDtypeStruct((_ROWS, _N), jnp.int32),
    )(x)


def _sc_oscarmax(x, rank, recip):
    mesh = plsc.VectorSubcoreMesh(core_axis_name="c", subcore_axis_name="s")
    nv = _N // 16

    @functools.partial(
        pl.kernel,
        mesh=mesh,
        out_type=jax.ShapeDtypeStruct((_ROWS, _N), jnp.float32),
        compiler_params=pltpu.CompilerParams(
            use_tc_tiling_on_sc=False, needs_layout_passes=False),
        scratch_types=[
            pltpu.VMEM((_N,), jnp.float32),
            pltpu.VMEM((_N,), jnp.int32),
            pltpu.VMEM((_N,), jnp.float32),
            pltpu.VMEM((_N,), jnp.float32),
            pltpu.VMEM((_N,), jnp.float32),
            pltpu.VMEM((_N + 32,), jnp.int32),
            pltpu.VMEM((_N + 16,), jnp.float32),
            pltpu.VMEM((_N + 16,), jnp.float32),
            pltpu.VMEM((_N + 16,), jnp.float32),
            pltpu.VMEM((_N + 16,), jnp.float32),
            pltpu.VMEM((_N,), jnp.float32),
            pltpu.VMEM((_N,), jnp.float32),
            pltpu.VMEM((_N,), jnp.float32),
            pltpu.VMEM((_N + 16,), jnp.float32),
        ],
    )
    def k(x_hbm, rank_hbm, recip_hbm, out_hbm, v_ref, r_ref, a_ref, sg_ref,
          ca_ref, st_ref, rm_ref, rc_ref, pm_ref, pc_ref, y_ref, zs_ref,
          o_ref, recip_ref):
        wid = lax.axis_index("s") * 2 + lax.axis_index("c")
        lane = lax.broadcasted_iota(jnp.int32, (16,), 0)
        inf = jnp.float32(jnp.inf)

        def sload(ref, idx):
            return plsc.load_gather(ref, [jnp.full((16,), idx, jnp.int32)])[0]

        def sstore(ref, idx, val):
            plsc.store_scatter(ref, [jnp.full((16,), idx, jnp.int32)],
                               jnp.full((16,), val, ref.dtype))

        @pl.when(wid < _ROWS)
        def _():
            row = wid
            pltpu.sync_copy(x_hbm.at[row], v_ref)
            pltpu.sync_copy(rank_hbm.at[row], r_ref)
            pltpu.sync_copy(recip_hbm, recip_ref)

            def scat(p, npos):
                r = r_ref[pl.ds(p * 16, 16)]
                xv = v_ref[pl.ds(p * 16, 16)]
                plsc.store_scatter(a_ref, [r], jnp.abs(xv))
                plsc.store_scatter(sg_ref, [r], jnp.sign(xv))
                return npos + jnp.sum((xv > 0).astype(jnp.float32))

            npos = lax.fori_loop(0, nv, scat, jnp.float32(0.0), unroll=4)

            def runscan(p, carry):
                cA, nbrk = carry
                av = a_ref[pl.ds(p * 16, 16)]
                incl = plsc.cumsum(av) + cA
                ca_ref[pl.ds(p * 16, 16)] = incl
                gpos = lane + p * 16
                prev = plsc.load_gather(a_ref, [jnp.maximum(gpos - 1, 0)])
                prev = jnp.where(gpos == 0, inf, prev)
                brk = (prev - av) > jnp.float32(_BETA)
                bf = brk.astype(jnp.float32)
                binc = plsc.cumsum(bf)
                tgt = nbrk + (binc - bf).astype(jnp.int32)
                plsc.store_scatter(st_ref, [tgt], gpos, mask=brk)
                return (incl[15], nbrk + binc[15].astype(jnp.int32))

            _, nrun = lax.fori_loop(0, nv, runscan, (jnp.float32(0.0),
                                                     jnp.int32(0)), unroll=2)
            sstore(st_ref, nrun, jnp.int32(_N))

            def runstat(q, _):
                base = q * 16
                idx = lane + base
                valid = idx < nrun
                cidx = jnp.where(valid, idx, 0)
                b = plsc.load_gather(st_ref, [cidx])
                e = plsc.load_gather(st_ref, [cidx + 1])
                bf = b.astype(jnp.float32)
                ef = e.astype(jnp.float32)
                cb = jnp.where(b == 0, 0.0,
                               plsc.load_gather(ca_ref,
                                                [jnp.maximum(b - 1, 0)]))
                ce = plsc.load_gather(ca_ref, [jnp.maximum(e - 1, 0)])
                cnt = ef - bf
                sum_a = ce - cb
                sum_w = _BETA * (cnt * float(_N - 1)
                                 - (bf + ef - 1.0) * cnt * 0.5)
                rcp = plsc.load_gather(
                    recip_ref, [jnp.where(valid, e - b, 1)])
                rm_ref[pl.ds(base, 16)] = (sum_a - sum_w) * rcp
                rc_ref[pl.ds(base, 16)] = jnp.where(valid, cnt, 0.0)
                return 0

            nvq = (nrun + 15) >> 4
            lax.fori_loop(0, nvq, runstat, 0)

            def pav(q, carry):
                d, tm, tc = carry
                cm = sload(rm_ref, q)
                cc = sload(rc_ref, q)

                def cond(st):
                    _d, ttm, _tc, m, _c = st
                    return ttm <= m

                def merge(st):
                    dd, ttm, ttc, m, c = st
                    c2 = ttc + c
                    m2 = (ttm * ttc + m * c) * sload(
                        recip_ref, c2.astype(jnp.int32))
                    dd = dd - 1
                    return (dd, sload(pm_ref, dd - 1), sload(pc_ref, dd - 1),
                            m2, c2)

                d, tm, tc, cm, cc = lax.while_loop(
                    cond, merge, (d, tm, tc, cm, cc))
                sstore(pm_ref, d - 1, tm)
                sstore(pc_ref, d - 1, tc)
                return (d + 1, cm, cc)

            d, tm, tc = lax.fori_loop(
                0, nrun, pav, (jnp.int32(1), inf, jnp.float32(1.0)))
            sstore(pm_ref, d - 1, tm)
            sstore(pc_ref, d - 1, tc)

            def init_y(p, _):
                y_ref[pl.ds(p * 16, 16)] = jnp.full((16,), inf, jnp.float32)
                return 0

            lax.fori_loop(0, nv, init_y, 0, unroll=8)

            def scatter_pools(p, start_carry):
                slot = lane + p * 16
                valid = jnp.logical_and(slot >= 1, slot < d)
                pcv = jnp.where(valid, pc_ref[pl.ds(p * 16, 16)], 0.0)
                pmv = pm_ref[pl.ds(p * 16, 16)]
                incl = plsc.cumsum(pcv)
                starts = (start_carry + incl - pcv).astype(jnp.int32)
                plsc.store_scatter(y_ref, [starts], pmv, mask=valid)
                return start_carry + incl[15]

            lax.fori_loop(0, (d + 15) >> 4, scatter_pools, jnp.float32(0.0))

            def fill(p, neg_carry):
                yv = y_ref[pl.ds(p * 16, 16)]
                m = jnp.maximum(plsc.cummax(-yv), neg_carry)
                y_ref[pl.ds(p * 16, 16)] = -m
                return m[15]

            lax.fori_loop(0, nv, fill, -inf, unroll=2)

            def build_zs(p, carry):
                cpos, czer, cneg = carry
                yc = jnp.maximum(y_ref[pl.ds(p * 16, 16)], 0.0)
                sgv = sg_ref[pl.ds(p * 16, 16)]
                fp = (sgv > 0).astype(jnp.float32)
                fz = (sgv == 0).astype(jnp.float32)
                fn = (sgv < 0).astype(jnp.float32)
                ip_ = plsc.cumsum(fp)
                iz = plsc.cumsum(fz)
                in_ = plsc.cumsum(fn)
                tp = cpos + (ip_ - fp)
                tz = npos + czer + (iz - fz)
                tn = float(_N - 1) - (cneg + (in_ - fn))
                tgt = (fp * tp + fz * tz + fn * tn).astype(jnp.int32)
                val = (fp - fn) * yc
                plsc.store_scatter(zs_ref, [tgt], val)
                return (cpos + ip_[15], czer + iz[15], cneg + in_[15])

            lax.fori_loop(0, nv, build_zs,
                          (jnp.float32(0.0), jnp.float32(0.0),
                           jnp.float32(0.0)), unroll=2)

            def smax(p, carry):
                cs, ssum, scnt = carry
                zv = zs_ref[pl.ds(p * 16, 16)]
                ics = plsc.cumsum(zv) + cs
                kk = (lane + p * 16 + 1).astype(jnp.float32)
                sup = (1.0 + kk * zv) > ics
                sf = sup.astype(jnp.float32)
                ssum = ssum + jnp.sum(jnp.where(sup, zv, 0.0))
                scnt = scnt + jnp.sum(sf)
                return (ics[15], ssum, scnt)

            _, ssum, scnt = lax.fori_loop(
                0, nv, smax, (jnp.float32(0.0), jnp.float32(0.0),
                              jnp.float32(0.0)), unroll=2)
            k_z = jnp.maximum(scnt, 1.0)
            tau = (ssum - 1.0) * sload(recip_ref, k_z.astype(jnp.int32))

            def outp(p, _):
                r = r_ref[pl.ds(p * 16, 16)]
                yv = jnp.maximum(plsc.load_gather(y_ref, [r]), 0.0)
                sg = jnp.sign(v_ref[pl.ds(p * 16, 16)])
                o_ref[pl.ds(p * 16, 16)] = jnp.maximum(sg * yv - tau, 0.0)
                return 0

            lax.fori_loop(0, nv, outp, 0, unroll=4)
            pltpu.sync_copy(o_ref, out_hbm.at[row])

    return k(x, rank, recip)


_RECIP = np.float32(1.0) / np.maximum(
    np.arange(_N + 16, dtype=np.float32), np.float32(1.0))


def kernel(x):
    rank = _ranks_tc(x)
    return _sc_oscarmax(x, rank, jnp.asarray(_RECIP))


# speedup vs baseline: 73.2060x; 1.0704x over previous
import functools

import numpy as np

import jax
import jax.numpy as jnp
from jax import lax
from jax.experimental import pallas as pl
from jax.experimental.pallas import tpu as pltpu
from jax.experimental.pallas import tpu_sc as plsc

_N = 2048
_ROWS = 8
_BETA = 1.0
_KCHUNK = 1024


def _rank_body(x_ref, out_ref):
    k_all = lax.bitcast_convert_type(jnp.abs(x_ref[...]), jnp.int32)
    kT = jnp.transpose(k_all)
    nchunk = _N // _KCHUNK
    jlt_diag = (lax.broadcasted_iota(jnp.int32, (_KCHUNK, _KCHUNK), 1)
                < lax.broadcasted_iota(jnp.int32, (_KCHUNK, _KCHUNK), 0)
                ).astype(jnp.int32)
    for r in range(_ROWS):
        kr = k_all[r:r + 1, :]
        kc = kT[:, r:r + 1]
        col_parts = []
        colacc = [None] * nchunk
        for i_ in range(nchunk):
            kcol = kc[i_ * _KCHUNK:(i_ + 1) * _KCHUNK, :]
            acc2d = None
            for j_ in range(i_, nchunk):
                krow = kr[:, j_ * _KCHUNK:(j_ + 1) * _KCHUNK]
                if j_ == i_:
                    t = ((krow + jlt_diag) > kcol).astype(jnp.float32)
                else:
                    t = (krow > kcol).astype(jnp.float32)
                acc2d = t if acc2d is None else acc2d + t
                if j_ > i_:
                    colacc[j_] = t if colacc[j_] is None else colacc[j_] + t
            col_parts.append(jnp.sum(acc2d, axis=1, keepdims=True))
        row_parts = [
            jnp.zeros((1, _KCHUNK), jnp.float32) if colacc[j_] is None
            else (float(j_ * _KCHUNK)
                  - jnp.sum(colacc[j_], axis=0, keepdims=True))
            for j_ in range(nchunk)
        ]
        col_full = jnp.concatenate(col_parts, axis=0)
        out_ref[r:r + 1, :] = (jnp.transpose(col_full)
                               + jnp.concatenate(row_parts, axis=1)
                               ).astype(jnp.int32)


def _ranks_tc(x):
    return pl.pallas_call(
        _rank_body,
        out_shape=jax.ShapeDtypeStruct((_ROWS, _N), jnp.int32),
    )(x)


def _sc_oscarmax(x, rank, recip):
    mesh = plsc.VectorSubcoreMesh(core_axis_name="c", subcore_axis_name="s")
    nv = _N // 16

    @functools.partial(
        pl.kernel,
        mesh=mesh,
        out_type=jax.ShapeDtypeStruct((_ROWS, _N), jnp.float32),
        compiler_params=pltpu.CompilerParams(
            use_tc_tiling_on_sc=False, needs_layout_passes=False),
        scratch_types=[
            pltpu.VMEM((_N,), jnp.float32),
            pltpu.VMEM((_N,), jnp.int32),
            pltpu.VMEM((_N,), jnp.float32),
            pltpu.VMEM((_N,), jnp.float32),
            pltpu.VMEM((_N,), jnp.float32),
            pltpu.VMEM((_N + 32,), jnp.int32),
            pltpu.VMEM((_N + 16,), jnp.float32),
            pltpu.VMEM((_N + 16,), jnp.float32),
            pltpu.VMEM((_N + 16,), jnp.float32),
            pltpu.VMEM((_N + 16,), jnp.float32),
            pltpu.VMEM((_N,), jnp.float32),
            pltpu.VMEM((_N,), jnp.float32),
            pltpu.VMEM((_N,), jnp.float32),
            pltpu.VMEM((_N + 16,), jnp.float32),
        ],
    )
    def k(x_hbm, rank_hbm, recip_hbm, out_hbm, v_ref, r_ref, a_ref, sg_ref,
          ca_ref, st_ref, rm_ref, rc_ref, pm_ref, pc_ref, y_ref, zs_ref,
          o_ref, recip_ref):
        wid = lax.axis_index("s") * 2 + lax.axis_index("c")
        lane = lax.broadcasted_iota(jnp.int32, (16,), 0)
        inf = jnp.float32(jnp.inf)

        def sload(ref, idx):
            return plsc.load_gather(ref, [jnp.full((16,), idx, jnp.int32)])[0]

        def sstore(ref, idx, val):
            plsc.store_scatter(ref, [jnp.full((16,), idx, jnp.int32)],
                               jnp.full((16,), val, ref.dtype))

        @pl.when(wid < _ROWS)
        def _():
            row = wid
            pltpu.sync_copy(x_hbm.at[row], v_ref)
            pltpu.sync_copy(rank_hbm.at[row], r_ref)
            pltpu.sync_copy(recip_hbm, recip_ref)

            def scat(p, npos):
                r = r_ref[pl.ds(p * 16, 16)]
                xv = v_ref[pl.ds(p * 16, 16)]
                plsc.store_scatter(a_ref, [r], jnp.abs(xv))
                plsc.store_scatter(sg_ref, [r], jnp.sign(xv))
                return npos + jnp.sum((xv > 0).astype(jnp.float32))

            npos = lax.fori_loop(0, nv, scat, jnp.float32(0.0), unroll=4)

            def runscan(p, carry):
                cA, nbrk = carry
                av = a_ref[pl.ds(p * 16, 16)]
                incl = plsc.cumsum(av) + cA
                ca_ref[pl.ds(p * 16, 16)] = incl
                gpos = lane + p * 16
                prev = plsc.load_gather(a_ref, [jnp.maximum(gpos - 1, 0)])
                prev = jnp.where(gpos == 0, inf, prev)
                brk = (prev - av) > jnp.float32(_BETA)
                bf = brk.astype(jnp.float32)
                binc = plsc.cumsum(bf)
                tgt = nbrk + (binc - bf).astype(jnp.int32)
                plsc.store_scatter(st_ref, [tgt], gpos, mask=brk)
                return (incl[15], nbrk + binc[15].astype(jnp.int32))

            _, nrun = lax.fori_loop(0, nv, runscan, (jnp.float32(0.0),
                                                     jnp.int32(0)), unroll=2)
            sstore(st_ref, nrun, jnp.int32(_N))

            def runstat(q, _):
                base = q * 16
                idx = lane + base
                valid = idx < nrun
                cidx = jnp.where(valid, idx, 0)
                b = plsc.load_gather(st_ref, [cidx])
                e = plsc.load_gather(st_ref, [cidx + 1])
                bf = b.astype(jnp.float32)
                ef = e.astype(jnp.float32)
                cb = jnp.where(b == 0, 0.0,
                               plsc.load_gather(ca_ref,
                                                [jnp.maximum(b - 1, 0)]))
                ce = plsc.load_gather(ca_ref, [jnp.maximum(e - 1, 0)])
                cnt = ef - bf
                sum_a = ce - cb
                sum_w = _BETA * (cnt * float(_N - 1)
                                 - (bf + ef - 1.0) * cnt * 0.5)
                rcp = plsc.load_gather(
                    recip_ref, [jnp.where(valid, e - b, 1)])
                rm_ref[pl.ds(base, 16)] = (sum_a - sum_w) * rcp
                rc_ref[pl.ds(base, 16)] = jnp.where(valid, cnt, 0.0)
                return 0

            nvq = (nrun + 15) >> 4
            lax.fori_loop(0, nvq, runstat, 0)

            def pav(q, carry):
                d, tm, tc = carry
                cm = sload(rm_ref, q)
                cc = sload(rc_ref, q)

                def cond(st):
                    _d, ttm, _tc, m, _c = st
                    return ttm <= m

                def merge(st):
                    dd, ttm, ttc, m, c = st
                    c2 = ttc + c
                    m2 = (ttm * ttc + m * c) * sload(
                        recip_ref, c2.astype(jnp.int32))
                    dd = dd - 1
                    return (dd, sload(pm_ref, dd - 1), sload(pc_ref, dd - 1),
                            m2, c2)

                d, tm, tc, cm, cc = lax.while_loop(
                    cond, merge, (d, tm, tc, cm, cc))
                sstore(pm_ref, d - 1, tm)
                sstore(pc_ref, d - 1, tc)
                return (d + 1, cm, cc)

            d, tm, tc = lax.fori_loop(
                0, nrun, pav, (jnp.int32(1), inf, jnp.float32(1.0)))
            sstore(pm_ref, d - 1, tm)
            sstore(pc_ref, d - 1, tc)

            def init_y(p, _):
                y_ref[pl.ds(p * 16, 16)] = jnp.full((16,), inf, jnp.float32)
                return 0

            lax.fori_loop(0, nv, init_y, 0, unroll=8)

            def scatter_pools(p, start_carry):
                slot = lane + p * 16
                valid = jnp.logical_and(slot >= 1, slot < d)
                pcv = jnp.where(valid, pc_ref[pl.ds(p * 16, 16)], 0.0)
                pmv = pm_ref[pl.ds(p * 16, 16)]
                incl = plsc.cumsum(pcv)
                starts = (start_carry + incl - pcv).astype(jnp.int32)
                plsc.store_scatter(y_ref, [starts], pmv, mask=valid)
                return start_carry + incl[15]

            lax.fori_loop(0, (d + 15) >> 4, scatter_pools, jnp.float32(0.0))

            def fill(p, neg_carry):
                yv = y_ref[pl.ds(p * 16, 16)]
                m = jnp.maximum(plsc.cummax(-yv), neg_carry)
                y_ref[pl.ds(p * 16, 16)] = -m
                return m[15]

            lax.fori_loop(0, nv, fill, -inf, unroll=2)

            def build_zs(p, carry):
                cpos, czer, cneg = carry
                yc = jnp.maximum(y_ref[pl.ds(p * 16, 16)], 0.0)
                sgv = sg_ref[pl.ds(p * 16, 16)]
                fp = (sgv > 0).astype(jnp.float32)
                fz = (sgv == 0).astype(jnp.float32)
                fn = (sgv < 0).astype(jnp.float32)
                ip_ = plsc.cumsum(fp)
                iz = plsc.cumsum(fz)
                in_ = plsc.cumsum(fn)
                tp = cpos + (ip_ - fp)
                tz = npos + czer + (iz - fz)
                tn = float(_N - 1) - (cneg + (in_ - fn))
                tgt = (fp * tp + fz * tz + fn * tn).astype(jnp.int32)
                val = (fp - fn) * yc
                plsc.store_scatter(zs_ref, [tgt], val)
                return (cpos + ip_[15], czer + iz[15], cneg + in_[15])

            lax.fori_loop(0, nv, build_zs,
                          (jnp.float32(0.0), jnp.float32(0.0),
                           jnp.float32(0.0)), unroll=2)

            def smax(p, carry):
                cs, ssum, scnt = carry
                zv = zs_ref[pl.ds(p * 16, 16)]
                ics = plsc.cumsum(zv) + cs
                kk = (lane + p * 16 + 1).astype(jnp.float32)
                sup = (1.0 + kk * zv) > ics
                sf = sup.astype(jnp.float32)
                ssum = ssum + jnp.sum(jnp.where(sup, zv, 0.0))
                scnt = scnt + jnp.sum(sf)
                return (ics[15], ssum, scnt)

            _, ssum, scnt = lax.fori_loop(
                0, nv, smax, (jnp.float32(0.0), jnp.float32(0.0),
                              jnp.float32(0.0)), unroll=2)
            k_z = jnp.maximum(scnt, 1.0)
            tau = (ssum - 1.0) * sload(recip_ref, k_z.astype(jnp.int32))

            def outp(p, _):
                r = r_ref[pl.ds(p * 16, 16)]
                yv = jnp.maximum(plsc.load_gather(y_ref, [r]), 0.0)
                sg = jnp.sign(v_ref[pl.ds(p * 16, 16)])
                o_ref[pl.ds(p * 16, 16)] = jnp.maximum(sg * yv - tau, 0.0)
                return 0

            lax.fori_loop(0, nv, outp, 0, unroll=4)
            pltpu.sync_copy(o_ref, out_hbm.at[row])

    return k(x, rank, recip)


_RECIP = np.float32(1.0) / np.maximum(
    np.arange(_N + 16, dtype=np.float32), np.float32(1.0))


def kernel(x):
    rank = _ranks_tc(x)
    return _sc_oscarmax(x, rank, jnp.asarray(_RECIP))


# speedup vs baseline: 74.6495x; 1.0197x over previous
import functools

import numpy as np

import jax
import jax.numpy as jnp
from jax import lax
from jax.experimental import pallas as pl
from jax.experimental.pallas import tpu as pltpu
from jax.experimental.pallas import tpu_sc as plsc

_N = 2048
_ROWS = 8
_BETA = 1.0
_KCHUNK = 1024


def _rank_body(x_ref, out_ref):
    k_all = lax.bitcast_convert_type(jnp.abs(x_ref[...]), jnp.int32)
    kT = jnp.transpose(k_all)
    nchunk = _N // _KCHUNK
    jlt_diag = (lax.broadcasted_iota(jnp.int32, (_KCHUNK, _KCHUNK), 1)
                < lax.broadcasted_iota(jnp.int32, (_KCHUNK, _KCHUNK), 0)
                ).astype(jnp.int32)
    for r in range(_ROWS):
        kr = k_all[r:r + 1, :]
        kc = kT[:, r:r + 1]
        col_parts = []
        colacc = [None] * nchunk
        for i_ in range(nchunk):
            kcol = kc[i_ * _KCHUNK:(i_ + 1) * _KCHUNK, :]
            acc2d = None
            for j_ in range(i_, nchunk):
                krow = kr[:, j_ * _KCHUNK:(j_ + 1) * _KCHUNK]
                if j_ == i_:
                    t = ((krow + jlt_diag) > kcol).astype(jnp.float32)
                else:
                    t = (krow > kcol).astype(jnp.float32)
                acc2d = t if acc2d is None else acc2d + t
                if j_ > i_:
                    colacc[j_] = t if colacc[j_] is None else colacc[j_] + t
            col_parts.append(jnp.sum(acc2d, axis=1, keepdims=True))
        row_parts = [
            jnp.zeros((1, _KCHUNK), jnp.float32) if colacc[j_] is None
            else (float(j_ * _KCHUNK)
                  - jnp.sum(colacc[j_], axis=0, keepdims=True))
            for j_ in range(nchunk)
        ]
        col_full = jnp.concatenate(col_parts, axis=0)
        out_ref[r:r + 1, :] = (jnp.transpose(col_full)
                               + jnp.concatenate(row_parts, axis=1)
                               ).astype(jnp.int32)


def _ranks_tc(x):
    return pl.pallas_call(
        _rank_body,
        out_shape=jax.ShapeDtypeStruct((_ROWS, _N), jnp.int32),
    )(x)


def _sc_oscarmax(x, rank, recip):
    mesh = plsc.VectorSubcoreMesh(core_axis_name="c", subcore_axis_name="s")
    nv = _N // 16

    @functools.partial(
        pl.kernel,
        mesh=mesh,
        out_type=jax.ShapeDtypeStruct((_ROWS, _N), jnp.float32),
        compiler_params=pltpu.CompilerParams(
            use_tc_tiling_on_sc=False, needs_layout_passes=False),
        scratch_types=[
            pltpu.VMEM((_N,), jnp.float32),
            pltpu.VMEM((_N,), jnp.int32),
            pltpu.VMEM((_N,), jnp.float32),
            pltpu.VMEM((_N,), jnp.float32),
            pltpu.VMEM((_N,), jnp.float32),
            pltpu.VMEM((_N + 32,), jnp.int32),
            pltpu.VMEM((_N + 16,), jnp.float32),
            pltpu.VMEM((_N + 16,), jnp.float32),
            pltpu.VMEM((_N + 16,), jnp.float32),
            pltpu.VMEM((_N + 16,), jnp.float32),
            pltpu.VMEM((_N,), jnp.float32),
            pltpu.VMEM((_N,), jnp.float32),
            pltpu.VMEM((_N,), jnp.float32),
            pltpu.VMEM((_N + 16,), jnp.float32),
            pltpu.SemaphoreType.DMA,
            pltpu.SemaphoreType.DMA,
            pltpu.SemaphoreType.DMA,
        ],
    )
    def k(x_hbm, rank_hbm, recip_hbm, out_hbm, v_ref, r_ref, a_ref, sg_ref,
          ca_ref, st_ref, rm_ref, rc_ref, pm_ref, pc_ref, y_ref, zs_ref,
          o_ref, recip_ref, sem1, sem2, sem3):
        wid = lax.axis_index("s") * 2 + lax.axis_index("c")
        lane = lax.broadcasted_iota(jnp.int32, (16,), 0)
        inf = jnp.float32(jnp.inf)

        def sload(ref, idx):
            return plsc.load_gather(ref, [jnp.full((16,), idx, jnp.int32)])[0]

        def sstore(ref, idx, val):
            plsc.store_scatter(ref, [jnp.full((16,), idx, jnp.int32)],
                               jnp.full((16,), val, ref.dtype))

        @pl.when(wid < _ROWS)
        def _():
            row = wid
            h1 = pltpu.async_copy(x_hbm.at[row], v_ref, sem1)
            h2 = pltpu.async_copy(rank_hbm.at[row], r_ref, sem2)
            h3 = pltpu.async_copy(recip_hbm, recip_ref, sem3)
            h1.wait()
            h2.wait()
            h3.wait()

            def scat(p, npos):
                r = r_ref[pl.ds(p * 16, 16)]
                xv = v_ref[pl.ds(p * 16, 16)]
                plsc.store_scatter(a_ref, [r], jnp.abs(xv))
                plsc.store_scatter(sg_ref, [r], jnp.sign(xv))
                return npos + jnp.sum((xv > 0).astype(jnp.float32))

            npos = lax.fori_loop(0, nv, scat, jnp.float32(0.0), unroll=4)

            def runscan(p, carry):
                cA, nbrk = carry
                av = a_ref[pl.ds(p * 16, 16)]
                incl = plsc.cumsum(av) + cA
                ca_ref[pl.ds(p * 16, 16)] = incl
                gpos = lane + p * 16
                prev = plsc.load_gather(a_ref, [jnp.maximum(gpos - 1, 0)])
                prev = jnp.where(gpos == 0, inf, prev)
                brk = (prev - av) > jnp.float32(_BETA)
                bf = brk.astype(jnp.float32)
                binc = plsc.cumsum(bf)
                tgt = nbrk + (binc - bf).astype(jnp.int32)
                plsc.store_scatter(st_ref, [tgt], gpos, mask=brk)
                return (incl[15], nbrk + binc[15].astype(jnp.int32))

            _, nrun = lax.fori_loop(0, nv, runscan, (jnp.float32(0.0),
                                                     jnp.int32(0)), unroll=2)
            sstore(st_ref, nrun, jnp.int32(_N))

            def runstat(q, _):
                base = q * 16
                idx = lane + base
                valid = idx < nrun
                cidx = jnp.where(valid, idx, 0)
                b = plsc.load_gather(st_ref, [cidx])
                e = plsc.load_gather(st_ref, [cidx + 1])
                bf = b.astype(jnp.float32)
                ef = e.astype(jnp.float32)
                cb = jnp.where(b == 0, 0.0,
                               plsc.load_gather(ca_ref,
                                                [jnp.maximum(b - 1, 0)]))
                ce = plsc.load_gather(ca_ref, [jnp.maximum(e - 1, 0)])
                cnt = ef - bf
                sum_a = ce - cb
                sum_w = _BETA * (cnt * float(_N - 1)
                                 - (bf + ef - 1.0) * cnt * 0.5)
                rcp = plsc.load_gather(
                    recip_ref, [jnp.where(valid, e - b, 1)])
                rm_ref[pl.ds(base, 16)] = (sum_a - sum_w) * rcp
                rc_ref[pl.ds(base, 16)] = jnp.where(valid, cnt, 0.0)
                return 0

            nvq = (nrun + 15) >> 4
            lax.fori_loop(0, nvq, runstat, 0)

            def pav(q, carry):
                d, tm, tc = carry
                cm = sload(rm_ref, q)
                cc = sload(rc_ref, q)

                def cond(st):
                    _d, ttm, _tc, m, _c = st
                    return ttm <= m

                def merge(st):
                    dd, ttm, ttc, m, c = st
                    c2 = ttc + c
                    m2 = (ttm * ttc + m * c) * sload(
                        recip_ref, c2.astype(jnp.int32))
                    dd = dd - 1
                    return (dd, sload(pm_ref, dd - 1), sload(pc_ref, dd - 1),
                            m2, c2)

                d, tm, tc, cm, cc = lax.while_loop(
                    cond, merge, (d, tm, tc, cm, cc))
                sstore(pm_ref, d - 1, tm)
                sstore(pc_ref, d - 1, tc)
                return (d + 1, cm, cc)

            d, tm, tc = lax.fori_loop(
                0, nrun, pav, (jnp.int32(1), inf, jnp.float32(1.0)))
            sstore(pm_ref, d - 1, tm)
            sstore(pc_ref, d - 1, tc)

            def init_y(p, _):
                y_ref[pl.ds(p * 16, 16)] = jnp.full((16,), inf, jnp.float32)
                return 0

            lax.fori_loop(0, nv, init_y, 0, unroll=8)

            def scatter_pools(p, start_carry):
                slot = lane + p * 16
                valid = jnp.logical_and(slot >= 1, slot < d)
                pcv = jnp.where(valid, pc_ref[pl.ds(p * 16, 16)], 0.0)
                pmv = pm_ref[pl.ds(p * 16, 16)]
                incl = plsc.cumsum(pcv)
                starts = (start_carry + incl - pcv).astype(jnp.int32)
                plsc.store_scatter(y_ref, [starts], pmv, mask=valid)
                return start_carry + incl[15]

            lax.fori_loop(0, (d + 15) >> 4, scatter_pools, jnp.float32(0.0))

            def fill(p, neg_carry):
                yv = y_ref[pl.ds(p * 16, 16)]
                m = jnp.maximum(plsc.cummax(-yv), neg_carry)
                y_ref[pl.ds(p * 16, 16)] = -m
                return m[15]

            lax.fori_loop(0, nv, fill, -inf, unroll=2)

            def build_zs(p, carry):
                cpos, czer, cneg = carry
                yc = jnp.maximum(y_ref[pl.ds(p * 16, 16)], 0.0)
                sgv = sg_ref[pl.ds(p * 16, 16)]
                fp = (sgv > 0).astype(jnp.float32)
                fz = (sgv == 0).astype(jnp.float32)
                fn = (sgv < 0).astype(jnp.float32)
                ip_ = plsc.cumsum(fp)
                iz = plsc.cumsum(fz)
                in_ = plsc.cumsum(fn)
                tp = cpos + (ip_ - fp)
                tz = npos + czer + (iz - fz)
                tn = float(_N - 1) - (cneg + (in_ - fn))
                tgt = (fp * tp + fz * tz + fn * tn).astype(jnp.int32)
                val = (fp - fn) * yc
                plsc.store_scatter(zs_ref, [tgt], val)
                return (cpos + ip_[15], czer + iz[15], cneg + in_[15])

            lax.fori_loop(0, nv, build_zs,
                          (jnp.float32(0.0), jnp.float32(0.0),
                           jnp.float32(0.0)), unroll=2)

            def smax(p, carry):
                cs, ssum, scnt = carry
                zv = zs_ref[pl.ds(p * 16, 16)]
                ics = plsc.cumsum(zv) + cs
                kk = (lane + p * 16 + 1).astype(jnp.float32)
                sup = (1.0 + kk * zv) > ics
                sf = sup.astype(jnp.float32)
                ssum = ssum + jnp.sum(jnp.where(sup, zv, 0.0))
                scnt = scnt + jnp.sum(sf)
                return (ics[15], ssum, scnt)

            _, ssum, scnt = lax.fori_loop(
                0, nv, smax, (jnp.float32(0.0), jnp.float32(0.0),
                              jnp.float32(0.0)), unroll=2)
            k_z = jnp.maximum(scnt, 1.0)
            tau = (ssum - 1.0) * sload(recip_ref, k_z.astype(jnp.int32))

            def outp(p, _):
                r = r_ref[pl.ds(p * 16, 16)]
                yv = jnp.maximum(plsc.load_gather(y_ref, [r]), 0.0)
                sg = jnp.sign(v_ref[pl.ds(p * 16, 16)])
                o_ref[pl.ds(p * 16, 16)] = jnp.maximum(sg * yv - tau, 0.0)
                return 0

            lax.fori_loop(0, nv, outp, 0, unroll=4)
            pltpu.sync_copy(o_ref, out_hbm.at[row])

    return k(x, rank, recip)


_RECIP = np.float32(1.0) / np.maximum(
    np.arange(_N + 16, dtype=np.float32), np.float32(1.0))


def kernel(x):
    rank = _ranks_tc(x)
    return _sc_oscarmax(x, rank, jnp.asarray(_RECIP))
